# baseline (speedup 1.0000x reference)
import functools

import jax
import jax.numpy as jnp
from jax import lax
from jax.experimental import pallas as pl
from jax.experimental.pallas import tpu as pltpu
from jax.experimental.pallas import tpu_sc as plsc

MAXLEN = 50
DIM = 64
BATCH = 16384
VOCAB = 1000000
PADW = DIM

NC = 2
NS = 16
NW = NC * NS

CHUNK = 128
BBLK = BATCH // CHUNK
NCHUNK = MAXLEN * BBLK
CPW = NCHUNK // NW
BPW = BBLK // NW
LANES = 16
JB = CHUNK // LANES
NBUF = 4


def _body(xT, table, pos, out, idx_v, rows, stg, pos_v, gsems, ssems):
    wid = lax.axis_index("s") * NC + lax.axis_index("c")
    col0 = wid * BPW * CHUNK

    pltpu.sync_copy(xT.at[:, pl.ds(col0, BPW * CHUNK)], idx_v)
    pltpu.sync_copy(pos, pos_v)

    bvecs = [lax.iota(jnp.int32, LANES) + LANES * j for j in range(JB)]
    lane0 = jnp.full((LANES,), 0, jnp.int32)

    def start_gather(s, j, b):
        idx = idx_v.at[s, pl.ds(j * CHUNK, CHUNK)]
        pltpu.async_copy(table.at[idx], rows[b], gsems[b])

    def wait_gather(b):
        pltpu.make_async_copy(
            table.at[idx_v.at[0, pl.ds(0, CHUNK)]], rows[b], gsems[b]
        ).wait()

    def start_store(s, j, b):
        dst = out.at[s, :, col0 // CHUNK + j, :, :]
        pltpu.async_copy(stg[b].at[:, :, pl.ds(0, CHUNK)], dst, ssems[b])

    def wait_store(b):
        pltpu.make_async_copy(
            stg[b].at[:, :, pl.ds(0, CHUNK)], out.at[0, :, 0, :, :], ssems[b]
        ).wait()

    dcol = [
        ((lax.iota(jnp.int32, LANES) + LANES * c) // 8,
         lax.rem(lax.iota(jnp.int32, LANES) + LANES * c, 8))
        for c in range(DIM // LANES)
    ]

    def transpose_add(s, b):
        rows_f = rows[b]
        for c in range(DIM // LANES):
            drv, slv = dcol[c]
            pval = pos_v[s, pl.ds(c * LANES, LANES)]
            for bb in range(CHUNK):
                v = rows_f[bb, pl.ds(c * LANES, LANES)] + pval
                plsc.store_scatter(stg[b], [drv, slv, lane0 + bb], v)

    def coords(g):
        return g // BPW, lax.rem(g, BPW)

    s0, j0 = coords(0)
    start_gather(s0, j0, 0)
    s1, j1 = coords(1)
    start_gather(s1, j1, 1)

    def block_body(i, carry):
        for k in range(NBUF):
            g = i * NBUF + k
            b = k

            @pl.when(g < CPW - 2)
            def _():
                sn, jn = coords(g + 2)
                start_gather(sn, jn, (k + 2) % NBUF)

            wait_gather(b)

            @pl.when(g >= NBUF)
            def _():
                wait_store(b)

            s, j = coords(g)
            transpose_add(s, b)
            start_store(s, j, b)
        return carry

    lax.fori_loop(0, CPW // NBUF, block_body, 0)

    for e in range(NBUF):
        wait_store((CPW - NBUF + e) % NBUF)


@functools.partial(
    pl.kernel,
    mesh=plsc.VectorSubcoreMesh(core_axis_name="c", subcore_axis_name="s"),
    out_type=jax.ShapeDtypeStruct((MAXLEN, DIM // 8, BBLK, 8, CHUNK), jnp.float32),
    scratch_types=[
        pltpu.VMEM((MAXLEN, BPW * CHUNK), jnp.int32),
        [pltpu.VMEM((CHUNK, PADW), jnp.float32) for _ in range(NBUF)],
        [pltpu.VMEM((DIM // 8, 8, CHUNK + 1), jnp.float32) for _ in range(NBUF)],
        pltpu.VMEM((DIM, DIM), jnp.float32),
        [pltpu.SemaphoreType.DMA for _ in range(NBUF)],
        [pltpu.SemaphoreType.DMA for _ in range(NBUF)],
    ],
    compiler_params=pltpu.CompilerParams(
        use_tc_tiling_on_sc=False, needs_layout_passes=False
    ),
)
def _sc_kernel(xT, table, pos, out, idx_v, rows, stg, pos_v, gsems, ssems):
    _body(xT, table, pos, out, idx_v, rows, stg, pos_v, gsems, ssems)


def kernel(x, token_table, pos_table):
    xT = x.astype(jnp.int32).T
    out5 = _sc_kernel(xT, token_table, pos_table)
    out = jnp.transpose(out5, (0, 1, 3, 2, 4)).reshape(MAXLEN, DIM, BATCH)
    return jnp.transpose(out, (2, 0, 1))


# speedup vs baseline: 1.0288x; 1.0288x over previous
import functools

import jax
import jax.numpy as jnp
from jax import lax
from jax.experimental import pallas as pl
from jax.experimental.pallas import tpu as pltpu
from jax.experimental.pallas import tpu_sc as plsc

MAXLEN = 50
DIM = 64
BATCH = 16384
VOCAB = 1000000
PADW = DIM

NC = 2
NS = 16
NW = NC * NS

CHUNK = 128
BBLK = BATCH // CHUNK
NCHUNK = MAXLEN * BBLK
CPW = NCHUNK // NW
BPW = BBLK // NW
LANES = 16
JB = CHUNK // LANES
NBUF = 2


def _body(xT, table, pos, out, idx_v, rows, stg, pos_v, gsems, ssems):
    wid = lax.axis_index("s") * NC + lax.axis_index("c")
    col0 = wid * BPW * CHUNK

    pltpu.sync_copy(xT.at[:, pl.ds(col0, BPW * CHUNK)], idx_v)
    pltpu.sync_copy(pos, pos_v)

    bvecs = [lax.iota(jnp.int32, LANES) + LANES * j for j in range(JB)]
    lane0 = jnp.full((LANES,), 0, jnp.int32)

    def start_gather(s, j, b):
        idx = idx_v.at[s, pl.ds(j * CHUNK, CHUNK)]
        pltpu.async_copy(table.at[idx], rows[b], gsems[b])

    def wait_gather(b):
        pltpu.make_async_copy(
            table.at[idx_v.at[0, pl.ds(0, CHUNK)]], rows[b], gsems[b]
        ).wait()

    def start_store(s, j, b):
        dst = out.at[s, :, col0 // CHUNK + j, :, :]
        pltpu.async_copy(stg[b].at[:, :, pl.ds(0, CHUNK)], dst, ssems[b])

    def wait_store(b):
        pltpu.make_async_copy(
            stg[b].at[:, :, pl.ds(0, CHUNK)], out.at[0, :, 0, :, :], ssems[b]
        ).wait()

    dcol = [
        ((lax.iota(jnp.int32, LANES) + LANES * c) // 8,
         lax.rem(lax.iota(jnp.int32, LANES) + LANES * c, 8))
        for c in range(DIM // LANES)
    ]

    def transpose_add(s, b):
        rows_f = rows[b]
        for c in range(DIM // LANES):
            drv, slv = dcol[c]
            pval = pos_v[s, pl.ds(c * LANES, LANES)]
            for bb in range(CHUNK):
                v = rows_f[bb, pl.ds(c * LANES, LANES)] + pval
                plsc.store_scatter(stg[b], [drv, slv, lane0 + bb], v)

    def coords(g):
        return g // BPW, lax.rem(g, BPW)

    s0, j0 = coords(0)
    start_gather(s0, j0, 0)

    def block_body(i, carry):
        for k in range(NBUF):
            g = i * NBUF + k
            b = k

            @pl.when(g < CPW - 1)
            def _():
                sn, jn = coords(g + 1)
                start_gather(sn, jn, (k + 1) % NBUF)

            wait_gather(b)

            @pl.when(g >= 2)
            def _():
                wait_store(b)

            s, j = coords(g)
            transpose_add(s, b)
            start_store(s, j, b)
        return carry

    lax.fori_loop(0, CPW // NBUF, block_body, 0)

    for e in range(NBUF):
        wait_store((CPW - NBUF + e) % NBUF)


@functools.partial(
    pl.kernel,
    mesh=plsc.VectorSubcoreMesh(core_axis_name="c", subcore_axis_name="s"),
    out_type=jax.ShapeDtypeStruct((MAXLEN, DIM // 8, BBLK, 8, CHUNK), jnp.float32),
    scratch_types=[
        pltpu.VMEM((MAXLEN, BPW * CHUNK), jnp.int32),
        [pltpu.VMEM((CHUNK, PADW), jnp.float32) for _ in range(NBUF)],
        [pltpu.VMEM((DIM // 8, 8, CHUNK + 1), jnp.float32) for _ in range(NBUF)],
        pltpu.VMEM((DIM, DIM), jnp.float32),
        [pltpu.SemaphoreType.DMA for _ in range(NBUF)],
        [pltpu.SemaphoreType.DMA for _ in range(NBUF)],
    ],
    compiler_params=pltpu.CompilerParams(
        use_tc_tiling_on_sc=False, needs_layout_passes=False
    ),
)
def _sc_kernel(xT, table, pos, out, idx_v, rows, stg, pos_v, gsems, ssems):
    _body(xT, table, pos, out, idx_v, rows, stg, pos_v, gsems, ssems)


def kernel(x, token_table, pos_table):
    xT = x.astype(jnp.int32).T
    out5 = _sc_kernel(xT, token_table, pos_table)
    out = jnp.transpose(out5, (0, 1, 3, 2, 4)).reshape(MAXLEN, DIM, BATCH)
    return jnp.transpose(out, (2, 0, 1))


# speedup vs baseline: 1.6909x; 1.6436x over previous
import functools

import jax
import jax.numpy as jnp
from jax import lax
from jax.experimental import pallas as pl
from jax.experimental.pallas import tpu as pltpu
from jax.experimental.pallas import tpu_sc as plsc

MAXLEN = 50
DIM = 64
BATCH = 16384
VOCAB = 1000000
PADW = DIM

NC = 2
NS = 16
NW = NC * NS

CHUNK = 128
BBLK = BATCH // CHUNK
NCHUNK = MAXLEN * BBLK
CPW = NCHUNK // NW
BPW = BBLK // NW
LANES = 16
JB = CHUNK // LANES
NBUF = 2


def _body(xT, table, pos, out, idx_v, rows, stg, pos_v, gsems, ssems):
    wid = lax.axis_index("s") * NC + lax.axis_index("c")
    col0 = wid * BPW * CHUNK

    pltpu.sync_copy(xT.at[:, pl.ds(col0, BPW * CHUNK)], idx_v)
    pltpu.sync_copy(pos, pos_v)

    bvecs = [lax.iota(jnp.int32, LANES) + LANES * j for j in range(JB)]
    lane0 = jnp.full((LANES,), 0, jnp.int32)

    def start_gather(s, j, b):
        idx = idx_v.at[s, pl.ds(j * CHUNK, CHUNK)]
        pltpu.async_copy(table.at[idx], rows[b], gsems[b])

    def wait_gather(b):
        pltpu.make_async_copy(
            table.at[idx_v.at[0, pl.ds(0, CHUNK)]], rows[b], gsems[b]
        ).wait()

    def start_store(s, j, b):
        dst = out.at[s, :, col0 // CHUNK + j, :, :]
        pltpu.async_copy(stg[b].at[:, :, pl.ds(0, CHUNK)], dst, ssems[b])

    def wait_store(b):
        pltpu.make_async_copy(
            stg[b].at[:, :, pl.ds(0, CHUNK)], out.at[0, :, 0, :, :], ssems[b]
        ).wait()

    dcol = [
        ((lax.iota(jnp.int32, LANES) + LANES * c) // 8,
         lax.rem(lax.iota(jnp.int32, LANES) + LANES * c, 8))
        for c in range(DIM // LANES)
    ]

    def transpose_add(s, b):
        rows_f = rows[b]
        pvals = [pos_v[s, pl.ds(c * LANES, LANES)] for c in range(DIM // LANES)]

        @plsc.parallel_loop(0, CHUNK, unroll=8)
        def _(bb):
            for c in range(DIM // LANES):
                drv, slv = dcol[c]
                v = rows_f[bb, pl.ds(c * LANES, LANES)] + pvals[c]
                plsc.store_scatter(stg[b], [drv, slv, lane0 + bb], v)

    def coords(g):
        return g // BPW, lax.rem(g, BPW)

    s0, j0 = coords(0)
    start_gather(s0, j0, 0)

    def block_body(i, carry):
        for k in range(NBUF):
            g = i * NBUF + k
            b = k

            @pl.when(g < CPW - 1)
            def _():
                sn, jn = coords(g + 1)
                start_gather(sn, jn, (k + 1) % NBUF)

            wait_gather(b)

            @pl.when(g >= 2)
            def _():
                wait_store(b)

            s, j = coords(g)
            transpose_add(s, b)
            start_store(s, j, b)
        return carry

    lax.fori_loop(0, CPW // NBUF, block_body, 0)

    for e in range(NBUF):
        wait_store((CPW - NBUF + e) % NBUF)


@functools.partial(
    pl.kernel,
    mesh=plsc.VectorSubcoreMesh(core_axis_name="c", subcore_axis_name="s"),
    out_type=jax.ShapeDtypeStruct((MAXLEN, DIM // 8, BBLK, 8, CHUNK), jnp.float32),
    scratch_types=[
        pltpu.VMEM((MAXLEN, BPW * CHUNK), jnp.int32),
        [pltpu.VMEM((CHUNK, PADW), jnp.float32) for _ in range(NBUF)],
        [pltpu.VMEM((DIM // 8, 8, CHUNK + 1), jnp.float32) for _ in range(NBUF)],
        pltpu.VMEM((DIM, DIM), jnp.float32),
        [pltpu.SemaphoreType.DMA for _ in range(NBUF)],
        [pltpu.SemaphoreType.DMA for _ in range(NBUF)],
    ],
    compiler_params=pltpu.CompilerParams(
        use_tc_tiling_on_sc=False, needs_layout_passes=False
    ),
)
def _sc_kernel(xT, table, pos, out, idx_v, rows, stg, pos_v, gsems, ssems):
    _body(xT, table, pos, out, idx_v, rows, stg, pos_v, gsems, ssems)


def kernel(x, token_table, pos_table):
    xT = x.astype(jnp.int32).T
    out5 = _sc_kernel(xT, token_table, pos_table)
    out = jnp.transpose(out5, (0, 1, 3, 2, 4)).reshape(MAXLEN, DIM, BATCH)
    return jnp.transpose(out, (2, 0, 1))
